# baseline (speedup 1.0000x reference)
import functools

import jax
import jax.numpy as jnp
from jax import lax
from jax.experimental import pallas as pl
from jax.experimental.pallas import tpu as pltpu
from jax.experimental.pallas import tpu_sc as plsc

BATCH = 16
SEQ = 50
G2 = 256
D = 64
NW = 32
SPW = BATCH * SEQ // NW
CH = 128
NCH = SPW * 2
NBUF = 4
LANES = 16


@functools.partial(
    pl.kernel,
    out_type=jax.ShapeDtypeStruct((BATCH, SEQ, G2, D), jnp.float32),
    mesh=plsc.VectorSubcoreMesh(core_axis_name="c", subcore_axis_name="s"),
    scratch_types=[
        pltpu.VMEM((NBUF, CH), jnp.int32),
        pltpu.VMEM((NBUF, CH, D), jnp.float32),
        pltpu.VMEM((G2, D), jnp.float32),
    ] + [pltpu.SemaphoreType.DMA] * (2 * NBUF),
    compiler_params=pltpu.CompilerParams(use_tc_tiling_on_sc=False),
)
def _embed_sc(idx_hbm, tab_hbm, pos_hbm, out_hbm, idx_v, rows_v, pos_v, *sems):
    sg = sems[:NBUF]
    ss = sems[NBUF:]
    wid = lax.axis_index("s") * 2 + lax.axis_index("c")
    bb = wid // 2
    ss0 = (wid % 2) * (SEQ // 2)
    pltpu.sync_copy(pos_hbm, pos_v)

    def fire_gather(c, b):
        s_ix = ss0 + c // 2
        g0 = pl.multiple_of((c % 2) * CH, CH)
        pltpu.sync_copy(idx_hbm.at[bb, s_ix, pl.ds(g0, CH)], idx_v.at[b])
        pltpu.async_copy(tab_hbm.at[idx_v.at[b]], rows_v.at[b], sg[b])

    def wait_gather(b):
        pltpu.make_async_copy(tab_hbm.at[idx_v.at[b]], rows_v.at[b],
                              sg[b]).wait()

    def fire_scatter(c, b):
        s_ix = ss0 + c // 2
        g0 = pl.multiple_of((c % 2) * CH, CH)
        pltpu.async_copy(rows_v.at[b], out_hbm.at[bb, s_ix, pl.ds(g0, CH)],
                         ss[b])

    def wait_scatter(b):
        pltpu.make_async_copy(rows_v.at[b], out_hbm.at[bb, ss0, pl.ds(0, CH)],
                              ss[b]).wait()

    def do_add(b, parity):
        pb = parity * CH

        @plsc.parallel_loop(0, CH, step=1, unroll=8)
        def _(g):
            for d in range(D // LANES):
                sl = pl.ds(d * LANES, LANES)
                rows_v[b, g, sl] = rows_v[b, g, sl] + pos_v[pb + g, sl]

    for c in range(NBUF - 1):
        fire_gather(c, c)

    def ring_body(i, _):
        for b in range(NBUF):
            c = NBUF * i + b
            b3 = (b + NBUF - 1) % NBUF

            @pl.when(c + NBUF - 1 < NCH)
            def _():
                @pl.when(c >= 1)
                def _():
                    wait_scatter(b3)

                fire_gather(c + NBUF - 1, b3)

            wait_gather(b)
            do_add(b, b & 1)
            fire_scatter(c, b)
        return 0

    full = NCH // NBUF
    lax.fori_loop(0, full, ring_body, 0)
    for c in range(full * NBUF, NCH):
        b = c % NBUF
        wait_gather(b)
        do_add(b, c & 1)
        fire_scatter(c, b)
    for b in range(NBUF):
        wait_scatter(b)


def kernel(spatial_tokens, token_embed_weight, pos_embed):
    pos = pos_embed.reshape(G2, D)
    return _embed_sc(spatial_tokens.astype(jnp.int32), token_embed_weight, pos)
